# baseline (speedup 1.0000x reference)
import jax
import jax.numpy as jnp
from jax import lax
from jax.experimental import pallas as pl
from jax.experimental.pallas import tpu as pltpu
from jax.experimental.pallas import tpu_sc as plsc

_N = 50
_E = 800
_NP = 64
_ECH = _E // 16


def _full16(v, dtype=jnp.int32):
    return jnp.full((16,), v, dtype)


def _adj_body(ei_hbm, cnt_hbm, ei_v, cnt_v):
    wid = lax.axis_index("s") * 2 + lax.axis_index("c")

    @pl.when(wid == 0)
    def _():
        pltpu.sync_copy(ei_hbm, ei_v)
        zero = jnp.zeros((16,), jnp.float32)

        def zbody(c, _):
            cnt_v[pl.ds(c * 16, 16)] = zero
            return 0

        lax.fori_loop(0, (_N * _NP) // 16, zbody, 0)
        one = jnp.full((16,), 1.0, jnp.float32)

        def ebody(c, _):
            isrc = ei_v[pl.ds(c * 16, 16)]
            idst = ei_v[pl.ds(_E + c * 16, 16)]
            plsc.addupdate_scatter(cnt_v, [isrc * _NP + idst], one)
            plsc.addupdate_scatter(cnt_v, [idst * _NP + isrc], one)
            return 0

        lax.fori_loop(0, _ECH, ebody, 0, unroll=5)
        pltpu.sync_copy(cnt_v, cnt_hbm)


def _gat_body(ei_hbm, xt_hbm, wf_hbm, out_hbm,
              ei_v, xt_v, wf_v, ha_v, hb_v, hw_v,
              ssrc_v, sdst_v, denom_v, lexp_v, eexp_v):
    wid = lax.axis_index("s") * 2 + lax.axis_index("c")

    @pl.when(wid == 0)
    def _():
        pltpu.sync_copy(ei_hbm, ei_v)
        pltpu.sync_copy(xt_hbm, xt_v)
        pltpu.sync_copy(wf_hbm, wf_v)

        dims = [(5, 3)] + [(3, 3)] * 9
        off = 0
        for li, (di, do) in enumerate(dims):
            href = xt_v if li == 0 else (hb_v if li % 2 == 1 else ha_v)
            oref = hb_v if li % 2 == 0 else ha_v
            wsp = [plsc.load_gather(wf_v, [_full16(off + t)])
                   for t in range(di * do)]
            asp = [plsc.load_gather(wf_v, [_full16(off + di * do + t)])
                   for t in range(do)]
            dsp = [plsc.load_gather(wf_v, [_full16(off + di * do + do + t)])
                   for t in range(do)]
            bsp = [plsc.load_gather(wf_v, [_full16(off + di * do + 2 * do + t)])
                   for t in range(do)]
            off += di * do + 3 * do

            for c in range(4):
                sl = pl.ds(c * 16, 16)
                hcols = [href[pl.ds(k * _NP + c * 16, 16)] for k in range(di)]
                accs = []
                for j in range(do):
                    acc = hcols[0] * wsp[j]
                    for k in range(1, di):
                        acc = acc + hcols[k] * wsp[k * do + j]
                    hw_v[pl.ds(j * _NP + c * 16, 16)] = acc
                    accs.append(acc)
                ss = accs[0] * asp[0]
                sd = accs[0] * dsp[0]
                for j in range(1, do):
                    ss = ss + accs[j] * asp[j]
                    sd = sd + accs[j] * dsp[j]
                ssrc_v[sl] = ss
                sdst_v[sl] = sd
                el = ss + sd
                el = jnp.where(el >= 0, el, 0.2 * el)
                ee = jnp.exp(el)
                denom_v[sl] = ee
                lexp_v[sl] = ee

            def p1(c, _):
                isrc = ei_v[pl.ds(c * 16, 16)]
                idst = ei_v[pl.ds(_E + c * 16, 16)]
                e = (plsc.load_gather(ssrc_v, [isrc]) +
                     plsc.load_gather(sdst_v, [idst]))
                e = jnp.where(e >= 0, e, 0.2 * e)
                ee = jnp.exp(e)
                eexp_v[pl.ds(c * 16, 16)] = ee
                plsc.addupdate_scatter(denom_v, [idst], ee)
                return 0

            lax.fori_loop(0, _ECH, p1, 0, unroll=5)

            for c in range(4):
                sl = pl.ds(c * 16, 16)
                al = lexp_v[sl] / (denom_v[sl] + 1e-16)
                for j in range(do):
                    oref[pl.ds(j * _NP + c * 16, 16)] = (
                        al * hw_v[pl.ds(j * _NP + c * 16, 16)] + bsp[j])

            def p2(c, _):
                isrc = ei_v[pl.ds(c * 16, 16)]
                idst = ei_v[pl.ds(_E + c * 16, 16)]
                ee = eexp_v[pl.ds(c * 16, 16)]
                dg = plsc.load_gather(denom_v, [idst])
                alpha = ee / (dg + 1e-16)
                for j in range(do):
                    gj = plsc.load_gather(hw_v, [isrc + (j * _NP)])
                    plsc.addupdate_scatter(oref, [idst + (j * _NP)],
                                           alpha * gj)
                return 0

            lax.fori_loop(0, _ECH, p2, 0, unroll=5)

            if li < 9:
                for c in range(4):
                    for j in range(do):
                        sl = pl.ds(j * _NP + c * 16, 16)
                        oref[sl] = jnp.maximum(oref[sl], 0.0)

        pltpu.sync_copy(ha_v, out_hbm)


def _head_body(ht_ref, evecs_ref, mask_ref,
               Wq_ref, Wk_ref, Wv_ref, Wo_ref,
               bq_ref, bk_ref, bv_ref, bo_ref,
               A1_ref, b1_ref, A2_ref, b2_ref, A3_ref, b3_ref,
               cw_ref, cb_ref, res_ref, val_ref):
    hrows = [ht_ref[j:j + 1, 0:_N] for j in range(3)]
    Wq, Wk, Wv, Wo = Wq_ref[...], Wk_ref[...], Wv_ref[...], Wo_ref[...]
    bq = bq_ref[...].reshape(1, 3)
    bk = bk_ref[...].reshape(1, 3)
    bv = bv_ref[...].reshape(1, 3)
    bo = bo_ref[...].reshape(1, 3)
    eye = (lax.broadcasted_iota(jnp.int32, (_N, _N), 0) ==
           lax.broadcasted_iota(jnp.int32, (_N, _N), 1)).astype(jnp.float32)

    def proj_rows(Wm, bm):
        rows = []
        for i in range(3):
            acc = None
            for j in range(3):
                term = hrows[j] * Wm[i:i + 1, j:j + 1]
                acc = term if acc is None else acc + term
            rows.append(acc + bm[0:1, i:i + 1])
        return rows

    qrows = proj_rows(Wq, bq)
    krows = proj_rows(Wk, bk)
    vrows = proj_rows(Wv, bv)
    ocols = []
    for i in range(3):
        qcol = jnp.sum(eye * qrows[i], axis=1, keepdims=True)
        s = qcol * krows[i]
        m = jnp.max(s, axis=1, keepdims=True)
        ex = jnp.exp(s - m)
        attn = ex / jnp.sum(ex, axis=1, keepdims=True)
        ocols.append(jnp.sum(attn * vrows[i], axis=1, keepdims=True))
    fcols = []
    for i in range(3):
        acc = None
        for j in range(3):
            term = ocols[j] * Wo[i:i + 1, j:j + 1]
            acc = term if acc is None else acc + term
        fcols.append(acc + bo[0:1, i:i + 1])
    hmha = jnp.concatenate(fcols, axis=1)

    x2 = jnp.concatenate([hmha, evecs_ref[...][:, 1:_N]], axis=1)
    hp = lax.Precision.HIGHEST
    r = jnp.dot(x2, A1_ref[...][0:52, :], precision=hp,
                preferred_element_type=jnp.float32) + b1_ref[...].reshape(1, 16)
    r = jnp.maximum(r, 0.0)
    r = jnp.dot(r, A2_ref[...], precision=hp,
                preferred_element_type=jnp.float32) + b2_ref[...].reshape(1, 32)
    r = jnp.maximum(r, 0.0)
    r = jnp.dot(r, A3_ref[...], precision=hp,
                preferred_element_type=jnp.float32) + b3_ref[...].reshape(1, 1)
    maskcol = jnp.sum(eye * mask_ref[...].reshape(1, _N), axis=1, keepdims=True)
    res_ref[...] = jnp.sum(eye * (r * maskcol), axis=0, keepdims=True)
    rc = r * cw_ref[0:1, 0:1] + cb_ref[...].reshape(1, 1)
    val_ref[...] = jnp.sum(rc, axis=0, keepdims=True) / float(_N)


def kernel(x, edge_index, mask, params):
    gat = params['gat']
    mha = params['mha']
    actor = params['actor']
    critic = params['critic']
    f32 = jnp.float32
    mesh = plsc.VectorSubcoreMesh(core_axis_name="c", subcore_axis_name="s")

    ei_flat = edge_index.reshape(2 * _E)

    scparams = pltpu.CompilerParams(needs_layout_passes=False)
    adj = pl.kernel(
        _adj_body, mesh=mesh, compiler_params=scparams,
        out_type=jax.ShapeDtypeStruct((_N * _NP,), f32),
        scratch_types=[pltpu.VMEM((2 * _E,), jnp.int32),
                       pltpu.VMEM((_N * _NP,), f32)],
    )
    cnt = adj(ei_flat)

    xt = jnp.zeros((5, _NP), f32).at[:, :_N].set(x.T).reshape(5 * _NP)
    wparts = []
    for g in gat:
        wparts += [g['W'].reshape(-1), g['a_src'], g['a_dst'], g['b']]
    wf = jnp.concatenate(wparts)
    wf_pad = jnp.zeros((208,), f32).at[:wf.shape[0]].set(wf)

    gatk = pl.kernel(
        _gat_body, mesh=mesh, compiler_params=scparams,
        out_type=jax.ShapeDtypeStruct((3 * _NP,), f32),
        scratch_types=[pltpu.VMEM((2 * _E,), jnp.int32),
                       pltpu.VMEM((5 * _NP,), f32),
                       pltpu.VMEM((208,), f32),
                       pltpu.VMEM((3 * _NP,), f32),
                       pltpu.VMEM((3 * _NP,), f32),
                       pltpu.VMEM((3 * _NP,), f32),
                       pltpu.VMEM((_NP,), f32),
                       pltpu.VMEM((_NP,), f32),
                       pltpu.VMEM((_NP,), f32),
                       pltpu.VMEM((_NP,), f32),
                       pltpu.VMEM((_E,), f32)],
    )
    ht = gatk(ei_flat, xt, wf_pad).reshape(3, _NP)

    asym = (cnt.reshape(_N, _NP)[:, :_N] > 0).astype(f32)
    deg = asym.sum(axis=1)
    dinv = jnp.where(deg > 0, 1.0 / jnp.sqrt(jnp.maximum(deg, 1e-12)), 0.0)
    Lm = jnp.eye(_N, dtype=f32) - (dinv[:, None] * asym) * dinv[None, :]
    _, evecs = jnp.linalg.eigh(Lm)

    res, val = pl.pallas_call(
        _head_body,
        out_shape=[jax.ShapeDtypeStruct((1, _N), f32),
                   jax.ShapeDtypeStruct((1, 1), f32)],
    )(ht, evecs, mask,
      mha['Wq'], mha['Wk'], mha['Wv'], mha['Wo'],
      mha['bq'], mha['bk'], mha['bv'], mha['bo'],
      actor['A1'], actor['b1'], actor['A2'], actor['b2'],
      actor['A3'], actor['b3'], critic['cw'], critic['cb'])

    return res.reshape(_N), val.reshape(())


# speedup vs baseline: 1.0648x; 1.0648x over previous
import jax
import jax.numpy as jnp
from jax import lax
from jax.experimental import pallas as pl
from jax.experimental.pallas import tpu as pltpu
from jax.experimental.pallas import tpu_sc as plsc

_N = 50
_E = 800
_NP = 64
_ECH = _E // 16
_DIMS = [(5, 3)] + [(3, 3)] * 9


def _full16(v, dtype=jnp.int32):
    return jnp.full((16,), v, dtype)


def _adj_body(ei_hbm, cnt_hbm, ei_v, cnt_v):
    wid = lax.axis_index("s") * 2 + lax.axis_index("c")

    @pl.when(wid == 0)
    def _():
        pltpu.sync_copy(ei_hbm, ei_v)
        zero = jnp.zeros((16,), jnp.float32)

        def zbody(c, _):
            cnt_v[pl.ds(c * 16, 16)] = zero
            return 0

        lax.fori_loop(0, (_N * _NP) // 16, zbody, 0, unroll=5)
        one = jnp.full((16,), 1.0, jnp.float32)

        def ebody(c, _):
            isrc = ei_v[0, pl.ds(c * 16, 16)]
            idst = ei_v[1, pl.ds(c * 16, 16)]
            plsc.addupdate_scatter(cnt_v, [isrc * _NP + idst], one)
            plsc.addupdate_scatter(cnt_v, [idst * _NP + isrc], one)
            return 0

        lax.fori_loop(0, _ECH, ebody, 0, unroll=5)
        pltpu.sync_copy(cnt_v, cnt_hbm)


def _gat_body(*refs):
    ei_hbm, x_hbm = refs[0], refs[1]
    par_hbm = refs[2:42]
    out_hbm = refs[42]
    ei_v, x_v, ha_v, hb_v, hw_v = refs[43:48]
    ssrc_v, sdst_v, denom_v, lexp_v, eexp_v = refs[48:53]
    par_v = refs[53:93]
    wid = lax.axis_index("s") * 2 + lax.axis_index("c")

    @pl.when(wid == 0)
    def _():
        pltpu.sync_copy(ei_hbm, ei_v)
        pltpu.sync_copy(x_hbm, x_v)
        for t in range(40):
            pltpu.sync_copy(par_hbm[t], par_v[t])

        lane = lax.broadcasted_iota(jnp.int32, (16,), 0)
        for li, (di, do) in enumerate(_DIMS):
            href = None if li == 0 else (hb_v if li % 2 == 1 else ha_v)
            oref = hb_v if li % 2 == 0 else ha_v
            W_v = par_v[4 * li]
            a_v = par_v[4 * li + 1]
            d_v = par_v[4 * li + 2]
            b_v = par_v[4 * li + 3]
            wsp = [[plsc.load_gather(W_v, [_full16(k), _full16(j)])
                    for j in range(do)] for k in range(di)]
            asp = [plsc.load_gather(a_v, [_full16(t)]) for t in range(do)]
            dsp = [plsc.load_gather(d_v, [_full16(t)]) for t in range(do)]
            bsp = [plsc.load_gather(b_v, [_full16(t)]) for t in range(do)]

            for c in range(4):
                sl = pl.ds(c * 16, 16)
                if li == 0:
                    nidx = jnp.minimum(c * 16 + lane, _N - 1)
                    hcols = [plsc.load_gather(x_v, [nidx, _full16(k)])
                             for k in range(di)]
                else:
                    hcols = [href[pl.ds(k * _NP + c * 16, 16)]
                             for k in range(di)]
                accs = []
                for j in range(do):
                    acc = hcols[0] * wsp[0][j]
                    for k in range(1, di):
                        acc = acc + hcols[k] * wsp[k][j]
                    hw_v[pl.ds(j * _NP + c * 16, 16)] = acc
                    accs.append(acc)
                ss = accs[0] * asp[0]
                sd = accs[0] * dsp[0]
                for j in range(1, do):
                    ss = ss + accs[j] * asp[j]
                    sd = sd + accs[j] * dsp[j]
                ssrc_v[sl] = ss
                sdst_v[sl] = sd
                el = ss + sd
                el = jnp.where(el >= 0, el, 0.2 * el)
                ee = jnp.exp(el)
                denom_v[sl] = ee
                lexp_v[sl] = ee

            def p1(c, _):
                isrc = ei_v[0, pl.ds(c * 16, 16)]
                idst = ei_v[1, pl.ds(c * 16, 16)]
                e = (plsc.load_gather(ssrc_v, [isrc]) +
                     plsc.load_gather(sdst_v, [idst]))
                e = jnp.where(e >= 0, e, 0.2 * e)
                ee = jnp.exp(e)
                eexp_v[pl.ds(c * 16, 16)] = ee
                plsc.addupdate_scatter(denom_v, [idst], ee)
                return 0

            lax.fori_loop(0, _ECH, p1, 0, unroll=5)

            for c in range(4):
                sl = pl.ds(c * 16, 16)
                al = lexp_v[sl] / (denom_v[sl] + 1e-16)
                for j in range(do):
                    oref[pl.ds(j * _NP + c * 16, 16)] = (
                        al * hw_v[pl.ds(j * _NP + c * 16, 16)] + bsp[j])

            def p2(c, _):
                isrc = ei_v[0, pl.ds(c * 16, 16)]
                idst = ei_v[1, pl.ds(c * 16, 16)]
                ee = eexp_v[pl.ds(c * 16, 16)]
                dg = plsc.load_gather(denom_v, [idst])
                alpha = ee / (dg + 1e-16)
                for j in range(do):
                    gj = plsc.load_gather(hw_v, [isrc + (j * _NP)])
                    plsc.addupdate_scatter(oref, [idst + (j * _NP)],
                                           alpha * gj)
                return 0

            lax.fori_loop(0, _ECH, p2, 0, unroll=5)

            if li < 9:
                for c in range(4):
                    for j in range(do):
                        sl = pl.ds(j * _NP + c * 16, 16)
                        oref[sl] = jnp.maximum(oref[sl], 0.0)

        pltpu.sync_copy(ha_v, out_hbm)


def _head_body(ht_ref, evecs_ref, mask_ref,
               Wq_ref, Wk_ref, Wv_ref, Wo_ref,
               bq_ref, bk_ref, bv_ref, bo_ref,
               A1_ref, b1_ref, A2_ref, b2_ref, A3_ref, b3_ref,
               cw_ref, cb_ref, res_ref, val_ref):
    hrows = [ht_ref[pl.ds(j * _NP, _N)].reshape(1, _N) for j in range(3)]
    Wq, Wk, Wv, Wo = Wq_ref[...], Wk_ref[...], Wv_ref[...], Wo_ref[...]
    bq = bq_ref[...].reshape(1, 3)
    bk = bk_ref[...].reshape(1, 3)
    bv = bv_ref[...].reshape(1, 3)
    bo = bo_ref[...].reshape(1, 3)
    eye = (lax.broadcasted_iota(jnp.int32, (_N, _N), 0) ==
           lax.broadcasted_iota(jnp.int32, (_N, _N), 1)).astype(jnp.float32)

    def proj_rows(Wm, bm):
        rows = []
        for i in range(3):
            acc = None
            for j in range(3):
                term = hrows[j] * Wm[i:i + 1, j:j + 1]
                acc = term if acc is None else acc + term
            rows.append(acc + bm[0:1, i:i + 1])
        return rows

    qrows = proj_rows(Wq, bq)
    krows = proj_rows(Wk, bk)
    vrows = proj_rows(Wv, bv)
    ocols = []
    for i in range(3):
        qcol = jnp.sum(eye * qrows[i], axis=1, keepdims=True)
        s = qcol * krows[i]
        m = jnp.max(s, axis=1, keepdims=True)
        ex = jnp.exp(s - m)
        attn = ex / jnp.sum(ex, axis=1, keepdims=True)
        ocols.append(jnp.sum(attn * vrows[i], axis=1, keepdims=True))
    fcols = []
    for i in range(3):
        acc = None
        for j in range(3):
            term = ocols[j] * Wo[i:i + 1, j:j + 1]
            acc = term if acc is None else acc + term
        fcols.append(acc + bo[0:1, i:i + 1])
    hmha = jnp.concatenate(fcols, axis=1)

    x2 = jnp.concatenate([hmha, evecs_ref[...][:, 1:_N]], axis=1)
    hp = lax.Precision.HIGHEST
    r = jnp.dot(x2, A1_ref[...][0:52, :], precision=hp,
                preferred_element_type=jnp.float32) + b1_ref[...].reshape(1, 16)
    r = jnp.maximum(r, 0.0)
    r = jnp.dot(r, A2_ref[...], precision=hp,
                preferred_element_type=jnp.float32) + b2_ref[...].reshape(1, 32)
    r = jnp.maximum(r, 0.0)
    r = jnp.dot(r, A3_ref[...], precision=hp,
                preferred_element_type=jnp.float32) + b3_ref[...].reshape(1, 1)
    maskcol = jnp.sum(eye * mask_ref[...].reshape(1, _N), axis=1, keepdims=True)
    res_ref[...] = jnp.sum(eye * (r * maskcol), axis=0, keepdims=True)
    rc = r * cw_ref[0:1, 0:1] + cb_ref[...].reshape(1, 1)
    val_ref[...] = jnp.sum(rc, axis=0, keepdims=True) / float(_N)


def kernel(x, edge_index, mask, params):
    gat = params['gat']
    mha = params['mha']
    actor = params['actor']
    critic = params['critic']
    f32 = jnp.float32
    mesh = plsc.VectorSubcoreMesh(core_axis_name="c", subcore_axis_name="s")
    scparams = pltpu.CompilerParams(needs_layout_passes=False)

    adj = pl.kernel(
        _adj_body, mesh=mesh, compiler_params=scparams,
        out_type=jax.ShapeDtypeStruct((_N * _NP,), f32),
        scratch_types=[pltpu.VMEM((2, _E), jnp.int32),
                       pltpu.VMEM((_N * _NP,), f32)],
    )
    cnt = adj(edge_index)

    par_args = []
    par_scratch = []
    for (di, do), g in zip(_DIMS, gat):
        par_args += [g['W'], g['a_src'], g['a_dst'], g['b']]
        par_scratch += [pltpu.VMEM((di, do), f32), pltpu.VMEM((do,), f32),
                        pltpu.VMEM((do,), f32), pltpu.VMEM((do,), f32)]

    gatk = pl.kernel(
        _gat_body, mesh=mesh, compiler_params=scparams,
        out_type=jax.ShapeDtypeStruct((3 * _NP,), f32),
        scratch_types=[pltpu.VMEM((2, _E), jnp.int32),
                       pltpu.VMEM((_N, 5), f32),
                       pltpu.VMEM((3 * _NP,), f32),
                       pltpu.VMEM((3 * _NP,), f32),
                       pltpu.VMEM((3 * _NP,), f32),
                       pltpu.VMEM((_NP,), f32),
                       pltpu.VMEM((_NP,), f32),
                       pltpu.VMEM((_NP,), f32),
                       pltpu.VMEM((_NP,), f32),
                       pltpu.VMEM((_E,), f32)] + par_scratch,
    )
    ht = gatk(edge_index, x, *par_args)

    asym = (cnt.reshape(_N, _NP)[:, :_N] > 0).astype(f32)
    deg = asym.sum(axis=1)
    dinv = jnp.where(deg > 0, 1.0 / jnp.sqrt(jnp.maximum(deg, 1e-12)), 0.0)
    Lm = jnp.eye(_N, dtype=f32) - (dinv[:, None] * asym) * dinv[None, :]
    _, evecs = jnp.linalg.eigh(Lm)

    res, val = pl.pallas_call(
        _head_body,
        out_shape=[jax.ShapeDtypeStruct((1, _N), f32),
                   jax.ShapeDtypeStruct((1, 1), f32)],
    )(ht, evecs, mask,
      mha['Wq'], mha['Wk'], mha['Wv'], mha['Wo'],
      mha['bq'], mha['bk'], mha['bv'], mha['bo'],
      actor['A1'], actor['b1'], actor['A2'], actor['b2'],
      actor['A3'], actor['b3'], critic['cw'], critic['cb'])

    return res.reshape(_N), val.reshape(())
